# Initial kernel scaffold; baseline (speedup 1.0000x reference)
import functools

import jax
import jax.numpy as jnp
from jax import lax
from jax.experimental import pallas as pl
from jax.experimental.pallas import tpu as pltpu
from jax.experimental.pallas import tpu_sc as plsc

D = 32
CH = 128
NC = 2
NS = 16
NW = NC * NS


@functools.partial(jax.jit, static_argnums=(2, 3))
def _sc_gather(idx2d, table, cpw, n_rows):
    mesh = plsc.VectorSubcoreMesh(core_axis_name="c", subcore_axis_name="s")

    @functools.partial(
        pl.kernel,
        out_type=jax.ShapeDtypeStruct((n_rows, D), jnp.float32),
        mesh=mesh,
        scratch_types=[
            pltpu.VMEM((cpw, CH), jnp.int32),
            pltpu.VMEM((CH, D), jnp.float32),
            pltpu.SemaphoreType.DMA,
        ],
    )
    def k(idx_hbm, table_hbm, out_hbm, idx_v, rows_v, sem):
        wid = lax.axis_index("s") * NC + lax.axis_index("c")
        pltpu.sync_copy(idx_hbm.at[pl.ds(wid * cpw, cpw)], idx_v)
        row_base = wid * (cpw * CH)

        def body(j, carry):
            pltpu.async_copy(table_hbm.at[idx_v.at[j]], rows_v, sem).wait()
            pltpu.sync_copy(rows_v, out_hbm.at[pl.ds(row_base + j * CH, CH)])
            return carry

        lax.fori_loop(0, cpw, body, 0)

    return k(idx2d, table)


def kernel(relation_ids, table):
    batch, seq = relation_ids.shape
    n_rows = batch * seq
    cpw = n_rows // (NW * CH)
    idx2d = relation_ids.reshape(n_rows // CH, CH).astype(jnp.int32)
    out = _sc_gather(idx2d, table.astype(jnp.float32), cpw, n_rows)
    return out.reshape(batch, seq, D)


# speedup vs baseline: 4.1968x; 4.1968x over previous
import functools

import jax
import jax.numpy as jnp
from jax import lax
from jax.experimental import pallas as pl
from jax.experimental.pallas import tpu as pltpu
from jax.experimental.pallas import tpu_sc as plsc

D = 32
CH = 128
NC = 2
NS = 16
NW = NC * NS


@functools.partial(jax.jit, static_argnums=(2, 3))
def _sc_gather(idx2d, table, cpw, n_rows):
    mesh = plsc.VectorSubcoreMesh(core_axis_name="c", subcore_axis_name="s")

    @functools.partial(
        pl.kernel,
        out_type=jax.ShapeDtypeStruct((n_rows, D), jnp.float32),
        mesh=mesh,
        scratch_types=[
            pltpu.VMEM((cpw, CH), jnp.int32),
            pltpu.VMEM((CH, D), jnp.float32),
            pltpu.SemaphoreType.DMA,
        ],
        compiler_params=pltpu.CompilerParams(use_tc_tiling_on_sc=False),
    )
    def k(idx_hbm, table_hbm, out_hbm, idx_v, rows_v, sem):
        wid = lax.axis_index("s") * NC + lax.axis_index("c")
        pltpu.sync_copy(idx_hbm.at[pl.ds(wid * cpw, cpw)], idx_v)
        row_base = wid * (cpw * CH)

        def body(j, carry):
            pltpu.async_copy(table_hbm.at[idx_v.at[j]], rows_v, sem).wait()
            pltpu.sync_copy(rows_v, out_hbm.at[pl.ds(row_base + j * CH, CH)])
            return carry

        lax.fori_loop(0, cpw, body, 0)

    return k(idx2d, table)


def kernel(relation_ids, table):
    batch, seq = relation_ids.shape
    n_rows = batch * seq
    cpw = n_rows // (NW * CH)
    idx2d = relation_ids.reshape(n_rows // CH, CH).astype(jnp.int32)
    out = _sc_gather(idx2d, table.astype(jnp.float32), cpw, n_rows)
    return out.reshape(batch, seq, D)


# speedup vs baseline: 5.1891x; 1.2364x over previous
import functools

import jax
import jax.numpy as jnp
from jax import lax
from jax.experimental import pallas as pl
from jax.experimental.pallas import tpu as pltpu
from jax.experimental.pallas import tpu_sc as plsc

D = 32
CH = 128
NC = 2
NS = 16
NW = NC * NS
NBUF = 4


@functools.partial(jax.jit, static_argnums=(2, 3))
def _sc_gather(idx2d, table, cpw, n_rows):
    mesh = plsc.VectorSubcoreMesh(core_axis_name="c", subcore_axis_name="s")
    ngrp = cpw // NBUF

    @functools.partial(
        pl.kernel,
        out_type=jax.ShapeDtypeStruct((n_rows, D), jnp.float32),
        mesh=mesh,
        scratch_types=(
            [pltpu.VMEM((cpw, CH), jnp.int32)]
            + [pltpu.VMEM((NBUF, CH, D), jnp.float32)]
            + [pltpu.SemaphoreType.DMA] * (2 * NBUF)
        ),
        compiler_params=pltpu.CompilerParams(use_tc_tiling_on_sc=False),
    )
    def k(idx_hbm, table_hbm, out_hbm, idx_v, rows_v, *sems):
        gsem, osem = sems[:NBUF], sems[NBUF:]
        wid = lax.axis_index("s") * NC + lax.axis_index("c")
        pltpu.sync_copy(idx_hbm.at[pl.ds(wid * cpw, cpw)], idx_v)
        row_base = wid * (cpw * CH)

        def start_gather(j, b):
            pltpu.async_copy(table_hbm.at[idx_v.at[j]], rows_v.at[b], gsem[b])

        def wait_gather(b):
            pltpu.make_async_copy(
                table_hbm.at[pl.ds(0, CH)], rows_v.at[b], gsem[b]
            ).wait()

        def start_out(j, b):
            return pltpu.async_copy(
                rows_v.at[b], out_hbm.at[pl.ds(row_base + j * CH, CH)], osem[b]
            )

        for b in range(NBUF):
            start_gather(b, b)

        def body(g, carry):
            outs = []
            for b in range(NBUF):
                wait_gather(b)
                outs.append(start_out(g * NBUF + b, b))
            for b in range(NBUF):
                outs[b].wait()
                start_gather((g + 1) * NBUF + b, b)
            return carry

        lax.fori_loop(0, ngrp - 1, body, 0)

        outs = []
        for b in range(NBUF):
            wait_gather(b)
            outs.append(start_out((ngrp - 1) * NBUF + b, b))
        for b in range(NBUF):
            outs[b].wait()

    return k(idx2d, table)


def kernel(relation_ids, table):
    batch, seq = relation_ids.shape
    n_rows = batch * seq
    cpw = n_rows // (NW * CH)
    idx2d = relation_ids.reshape(n_rows // CH, CH).astype(jnp.int32)
    out = _sc_gather(idx2d, table.astype(jnp.float32), cpw, n_rows)
    return out.reshape(batch, seq, D)


# speedup vs baseline: 5.2881x; 1.0191x over previous
import functools

import jax
import jax.numpy as jnp
from jax import lax
from jax.experimental import pallas as pl
from jax.experimental.pallas import tpu as pltpu
from jax.experimental.pallas import tpu_sc as plsc

D = 32
CH = 512
NC = 2
NS = 16
NW = NC * NS
NBUF = 5


@functools.partial(jax.jit, static_argnums=(2, 3))
def _sc_gather(idx2d, table, cpw, n_rows):
    mesh = plsc.VectorSubcoreMesh(core_axis_name="c", subcore_axis_name="s")
    ngrp = cpw // NBUF

    @functools.partial(
        pl.kernel,
        out_type=jax.ShapeDtypeStruct((n_rows, D), jnp.float32),
        mesh=mesh,
        scratch_types=(
            [pltpu.VMEM((cpw, CH), jnp.int32)]
            + [pltpu.VMEM((NBUF, CH, D), jnp.float32)]
            + [pltpu.SemaphoreType.DMA] * (2 * NBUF)
        ),
        compiler_params=pltpu.CompilerParams(use_tc_tiling_on_sc=False),
    )
    def k(idx_hbm, table_hbm, out_hbm, idx_v, rows_v, *sems):
        gsem, osem = sems[:NBUF], sems[NBUF:]
        wid = lax.axis_index("s") * NC + lax.axis_index("c")
        pltpu.sync_copy(idx_hbm.at[pl.ds(wid * cpw, cpw)], idx_v)
        row_base = wid * (cpw * CH)

        def start_gather(j, b):
            pltpu.async_copy(table_hbm.at[idx_v.at[j]], rows_v.at[b], gsem[b])

        def wait_gather(b):
            pltpu.make_async_copy(
                table_hbm.at[pl.ds(0, CH)], rows_v.at[b], gsem[b]
            ).wait()

        def start_out(j, b):
            return pltpu.async_copy(
                rows_v.at[b], out_hbm.at[pl.ds(row_base + j * CH, CH)], osem[b]
            )

        for b in range(NBUF):
            start_gather(b, b)

        def body(g, carry):
            outs = []
            for b in range(NBUF):
                wait_gather(b)
                outs.append(start_out(g * NBUF + b, b))
            for b in range(NBUF):
                outs[b].wait()
                start_gather((g + 1) * NBUF + b, b)
            return carry

        lax.fori_loop(0, ngrp - 1, body, 0)

        outs = []
        for b in range(NBUF):
            wait_gather(b)
            outs.append(start_out((ngrp - 1) * NBUF + b, b))
        for b in range(NBUF):
            outs[b].wait()

    return k(idx2d, table)


def kernel(relation_ids, table):
    batch, seq = relation_ids.shape
    n_rows = batch * seq
    cpw = n_rows // (NW * CH)
    idx2d = relation_ids.reshape(n_rows // CH, CH).astype(jnp.int32)
    out = _sc_gather(idx2d, table.astype(jnp.float32), cpw, n_rows)
    return out.reshape(batch, seq, D)
